# baseline (speedup 1.0000x reference)
import functools

import numpy as np
import jax
import jax.numpy as jnp
from jax import lax
from jax.experimental import pallas as pl
from jax.experimental.pallas import tpu as pltpu
from jax.experimental.pallas import tpu_sc as plsc

_MASK_RATIO = 0.75
_SEQ = 1024
_KEEP = _SEQ - int(_MASK_RATIO * _SEQ)
_BATCH = 64
_DIM = 768

_cache = {}

_FWD_PERM_STR = "955,914,121,753,617,480,35,577,130,263,799,942,557,148,895,883,197,793,410,649,398,934,973,605,45,520,1001,176,569,591,462,446,659,366,575,257,999,179,139,901,315,846,768,501,709,188,312,499,318,448,957,304,739,842,99,984,707,309,879,567,144,748,602,152,517,864,189,1005,582,780,487,552,750,544,516,325,31,1016,112,532,518,893,495,857,976,356,493,507,543,790,268,848,429,538,409,854,787,541,85,981,762,712,714,63,117,417,174,1020,565,441,962,509,1009,584,525,481,272,774,808,859,114,752,970,836,254,564,524,82,703,65,7,693,350,4,101,607,765,650,1021,816,463,928,452,444,102,78,163,708,1017,157,694,966,302,940,872,183,704,29,240,177,278,910,769,259,638,590,108,553,918,698,305,83,129,585,803,367,212,277,504,300,771,44,932,792,603,211,16,58,690,777,810,123,562,829,37,336,580,111,19,921,61,540,447,673,776,598,2,142,874,736,34,542,369,804,339,654,551,156,436,1010,5,996,911,461,589,415,90,885,715,706,363,514,175,167,284,379,251,600,110,619,904,72,155,1012,578,670,178,323,675,755,291,388,730,681,995,269,535,847,354,573,728,533,665,368,948,861,601,219,510,1023,153,30,275,705,42,186,342,406,468,1018,439,877,660,922,307,256,419,663,246,1006,3,643,362,380,327,903,393,70,729,566,378,400,920,794,926,271,592,969,588,979,522,614,488,311,947,67,612,273,223,422,39,56,630,274,192,169,349,998,218,785,195,476,173,900,975,245,241,959,69,943,383,646,811,80,22,820,571,924,906,6,321,199,345,118,235,766,54,442,479,423,266,721,77,425,147,18,1000,340,298,843,809,599,249,294,375,382,667,876,1015,819,815,10,938,635,570,689,699,977,751,11,987,234,53,236,455,641,1002,722,528,664,94,515,332,801,950,965,511,331,437,353,684,489,287,604,32,217,283,355,529,407,159,440,15,470,184,49,875,548,137,50,558,701,138,20,563,939,905,549,880,445,749,237,596,835,618,280,253,185,583,527,812,931,717,460,595,43,767,389,335,593,986,561,258,370,949,344,844,700,92,852,8,503,734,324,937,140,233,737,611,941,24,841,757,81,964,239,610,314,653,453,695,980,96,609,475,782,899,467,154,724,696,135,472,490,469,838,559,500,264,160,657,797,796,678,106,923,128,265,990,426,386,191,873,9,685,686,200,40,909,677,187,71,732,346,726,773,795,625,719,725,438,886,333,248,645,661,164,207,688,93,887,913,652,59,201,615,158,210,420,402,75,741,716,639,508,131,411,97,869,66,727,25,196,424,866,364,951,497,860,917,242,338,206,243,397,868,839,960,784,341,613,450,414,851,238,834,720,560,892,764,849,863,295,691,821,581,432,431,647,308,912,73,710,837,897,830,992,512,320,13,52,878,956,687,763,556,622,1013,642,631,491,203,289,702,303,202,915,255,194,88,833,672,250,337,62,230,894,150,261,674,330,919,262,209,586,760,132,357,87,76,806,198,896,486,968,862,626,60,946,759,740,735,244,457,651,807,813,47,392,374,597,827,276,683,770,33,79,606,1019,451,180,817,403,723,247,14,459,286,421,594,458,845,985,927,944,933,228,17,884,629,38,86,978,608,550,231,190,865,232,545,482,779,23,536,640,930,105,994,484,395,658,427,301,954,474,376,814,555,997,637,405,805,546,494,471,391,574,822,648,925,534,668,624,313,826,220,676,0,473,145,798,371,579,855,1011,213,226,381,133,281,758,41,64,572,416,982,21,655,443,161,576,744,279,285,988,916,679,166,124,116,449,26,802,165,168,193,57,208,713,181,89,789,146,182,936,126,125,297,1,115,28,972,991,113,731,692,853,775,530,628,225,361,351,537,465,172,1014,377,162,738,48,778,170,466,666,505,818,227,1008,974,36,252,890,502,492,521,119,151,385,828,682,989,306,662,791,832,120,372,1003,390,224,761,523,952,781,616,122,270,100,568,953,418,433,329,365,396,526,91,958,824,519,870,222,786,850,733,644,669,55,747,983,496,498,103,971,620,929,51,945,961,671,293,215,384,127,840,98,743,483,697,506,282,745,107,27,322,74,136,800,229,711,993,319,328,531,772,430,343,621,204,221,623,296,12,856,134,454,477,554,935,888,627,408,109,84,539,587,428,317,1022,788,513,358,394,299,205,831,171,288,143,632,68,267,908,216,783,435,547,149,485,434,141,464,334,404,634,104,882,352,95,907,387,858,871,316,891,881,742,718,963,633,214,290,754,867,1007,46,310,348,401,260,823,656,898,478,902,292,825,680,359,326,347,889,456,399,373,412,360,967,413,1004,636,756,746"


def _perm_consts():
    if "perm" not in _cache:
        fwd = np.array([int(v) for v in _FWD_PERM_STR.split(",")],
                       dtype=np.int32)
        bwd = np.argsort(fwd).astype(np.int32)
        flat_idx = (np.arange(_BATCH, dtype=np.int32)[:, None] * _SEQ
                    + fwd[None, :_KEEP]).reshape(-1)
        _cache["perm"] = (fwd, bwd, flat_idx)
    return _cache["perm"]


def _make_gather():
    if "gather" in _cache:
        return _cache["gather"]

    info = plsc.get_sparse_core_info()
    nw = info.num_cores * info.num_subcores
    n_rows = _BATCH * _KEEP
    rows_per_w = n_rows // nw
    chunk = 32
    nbuf = 5
    assert rows_per_w % chunk == 0
    n_chunks = rows_per_w // chunk
    mesh = plsc.VectorSubcoreMesh(core_axis_name="c", subcore_axis_name="s")

    @functools.partial(
        pl.kernel,
        mesh=mesh,
        out_type=jax.ShapeDtypeStruct((n_rows, _DIM), jnp.float32),
        scratch_types=[
            pltpu.VMEM((rows_per_w,), jnp.int32),
            pltpu.VMEM((nbuf, chunk, _DIM), jnp.float32),
            [pltpu.SemaphoreType.DMA] * nbuf,
            [pltpu.SemaphoreType.DMA] * nbuf,
        ],
    )
    def gather(table_hbm, idx_hbm, out_hbm, idx_v, rows_v, gsems, ssems):
        wid = lax.axis_index("s") * info.num_cores + lax.axis_index("c")
        base = wid * rows_per_w
        pltpu.sync_copy(idx_hbm.at[pl.ds(base, rows_per_w)], idx_v)

        def g_copy(g, b):
            return pltpu.make_async_copy(
                table_hbm.at[idx_v.at[pl.ds(g * chunk, chunk)]],
                rows_v.at[b], gsems[b])

        def s_copy(g, b):
            return pltpu.make_async_copy(
                rows_v.at[b], out_hbm.at[pl.ds(base + g * chunk, chunk)],
                ssems[b])

        for g in range(n_chunks):
            b = g % nbuf
            if g >= nbuf:
                s_copy(g - nbuf, b).wait()
            g_copy(g, b).start()
            if g >= 1:
                pb = (g - 1) % nbuf
                g_copy(g - 1, pb).wait()
                s_copy(g - 1, pb).start()
        last = n_chunks - 1
        g_copy(last, last % nbuf).wait()
        s_copy(last, last % nbuf).start()
        for g in range(max(0, n_chunks - nbuf), n_chunks):
            s_copy(g, g % nbuf).wait()

    _cache["gather"] = gather
    return gather


def _make_tc_gather(kept):
    kept = [int(v) for v in kept]

    def body(in_ref, out_ref, sem):
        copies = [
            pltpu.make_async_copy(in_ref.at[:, src], out_ref.at[:, i], sem)
            for i, src in enumerate(kept)
        ]
        for c in copies:
            c.start()
        for c in copies:
            c.wait()

    return pl.pallas_call(
        body,
        in_specs=[pl.BlockSpec(memory_space=pl.ANY)],
        out_specs=pl.BlockSpec(memory_space=pl.ANY),
        out_shape=jax.ShapeDtypeStruct((_BATCH, _KEEP, _DIM), jnp.float32),
        scratch_shapes=[pltpu.SemaphoreType.DMA],
    )


def kernel(input):
    assert input.shape == (_BATCH, _SEQ, _DIM), input.shape
    fwd, bwd, flat_idx = _perm_consts()
    table = input.reshape(_BATCH * _SEQ, _DIM)
    out = _make_gather()(table, jnp.asarray(flat_idx))
    out = out.reshape(_BATCH, _KEEP, _DIM)
    return (out, jnp.asarray(fwd), jnp.asarray(bwd))


# speedup vs baseline: 1.4759x; 1.4759x over previous
import functools

import numpy as np
import jax
import jax.numpy as jnp
from jax import lax
from jax.experimental import pallas as pl
from jax.experimental.pallas import tpu as pltpu
from jax.experimental.pallas import tpu_sc as plsc

_MASK_RATIO = 0.75
_SEQ = 1024
_KEEP = _SEQ - int(_MASK_RATIO * _SEQ)
_BATCH = 64
_DIM = 768

_cache = {}

_FWD_PERM_STR = "955,914,121,753,617,480,35,577,130,263,799,942,557,148,895,883,197,793,410,649,398,934,973,605,45,520,1001,176,569,591,462,446,659,366,575,257,999,179,139,901,315,846,768,501,709,188,312,499,318,448,957,304,739,842,99,984,707,309,879,567,144,748,602,152,517,864,189,1005,582,780,487,552,750,544,516,325,31,1016,112,532,518,893,495,857,976,356,493,507,543,790,268,848,429,538,409,854,787,541,85,981,762,712,714,63,117,417,174,1020,565,441,962,509,1009,584,525,481,272,774,808,859,114,752,970,836,254,564,524,82,703,65,7,693,350,4,101,607,765,650,1021,816,463,928,452,444,102,78,163,708,1017,157,694,966,302,940,872,183,704,29,240,177,278,910,769,259,638,590,108,553,918,698,305,83,129,585,803,367,212,277,504,300,771,44,932,792,603,211,16,58,690,777,810,123,562,829,37,336,580,111,19,921,61,540,447,673,776,598,2,142,874,736,34,542,369,804,339,654,551,156,436,1010,5,996,911,461,589,415,90,885,715,706,363,514,175,167,284,379,251,600,110,619,904,72,155,1012,578,670,178,323,675,755,291,388,730,681,995,269,535,847,354,573,728,533,665,368,948,861,601,219,510,1023,153,30,275,705,42,186,342,406,468,1018,439,877,660,922,307,256,419,663,246,1006,3,643,362,380,327,903,393,70,729,566,378,400,920,794,926,271,592,969,588,979,522,614,488,311,947,67,612,273,223,422,39,56,630,274,192,169,349,998,218,785,195,476,173,900,975,245,241,959,69,943,383,646,811,80,22,820,571,924,906,6,321,199,345,118,235,766,54,442,479,423,266,721,77,425,147,18,1000,340,298,843,809,599,249,294,375,382,667,876,1015,819,815,10,938,635,570,689,699,977,751,11,987,234,53,236,455,641,1002,722,528,664,94,515,332,801,950,965,511,331,437,353,684,489,287,604,32,217,283,355,529,407,159,440,15,470,184,49,875,548,137,50,558,701,138,20,563,939,905,549,880,445,749,237,596,835,618,280,253,185,583,527,812,931,717,460,595,43,767,389,335,593,986,561,258,370,949,344,844,700,92,852,8,503,734,324,937,140,233,737,611,941,24,841,757,81,964,239,610,314,653,453,695,980,96,609,475,782,899,467,154,724,696,135,472,490,469,838,559,500,264,160,657,797,796,678,106,923,128,265,990,426,386,191,873,9,685,686,200,40,909,677,187,71,732,346,726,773,795,625,719,725,438,886,333,248,645,661,164,207,688,93,887,913,652,59,201,615,158,210,420,402,75,741,716,639,508,131,411,97,869,66,727,25,196,424,866,364,951,497,860,917,242,338,206,243,397,868,839,960,784,341,613,450,414,851,238,834,720,560,892,764,849,863,295,691,821,581,432,431,647,308,912,73,710,837,897,830,992,512,320,13,52,878,956,687,763,556,622,1013,642,631,491,203,289,702,303,202,915,255,194,88,833,672,250,337,62,230,894,150,261,674,330,919,262,209,586,760,132,357,87,76,806,198,896,486,968,862,626,60,946,759,740,735,244,457,651,807,813,47,392,374,597,827,276,683,770,33,79,606,1019,451,180,817,403,723,247,14,459,286,421,594,458,845,985,927,944,933,228,17,884,629,38,86,978,608,550,231,190,865,232,545,482,779,23,536,640,930,105,994,484,395,658,427,301,954,474,376,814,555,997,637,405,805,546,494,471,391,574,822,648,925,534,668,624,313,826,220,676,0,473,145,798,371,579,855,1011,213,226,381,133,281,758,41,64,572,416,982,21,655,443,161,576,744,279,285,988,916,679,166,124,116,449,26,802,165,168,193,57,208,713,181,89,789,146,182,936,126,125,297,1,115,28,972,991,113,731,692,853,775,530,628,225,361,351,537,465,172,1014,377,162,738,48,778,170,466,666,505,818,227,1008,974,36,252,890,502,492,521,119,151,385,828,682,989,306,662,791,832,120,372,1003,390,224,761,523,952,781,616,122,270,100,568,953,418,433,329,365,396,526,91,958,824,519,870,222,786,850,733,644,669,55,747,983,496,498,103,971,620,929,51,945,961,671,293,215,384,127,840,98,743,483,697,506,282,745,107,27,322,74,136,800,229,711,993,319,328,531,772,430,343,621,204,221,623,296,12,856,134,454,477,554,935,888,627,408,109,84,539,587,428,317,1022,788,513,358,394,299,205,831,171,288,143,632,68,267,908,216,783,435,547,149,485,434,141,464,334,404,634,104,882,352,95,907,387,858,871,316,891,881,742,718,963,633,214,290,754,867,1007,46,310,348,401,260,823,656,898,478,902,292,825,680,359,326,347,889,456,399,373,412,360,967,413,1004,636,756,746"


def _perm_consts():
    if "perm" not in _cache:
        fwd = np.array([int(v) for v in _FWD_PERM_STR.split(",")],
                       dtype=np.int32)
        bwd = np.argsort(fwd).astype(np.int32)
        flat_idx = (np.arange(_BATCH, dtype=np.int32)[:, None] * _SEQ
                    + fwd[None, :_KEEP]).reshape(-1)
        _cache["perm"] = (fwd, bwd, flat_idx)
    return _cache["perm"]


def _make_gather():
    if "gather" in _cache:
        return _cache["gather"]

    info = plsc.get_sparse_core_info()
    nw = info.num_cores * info.num_subcores
    n_rows = _BATCH * _KEEP
    rows_per_w = n_rows // nw
    chunk = 32
    nbuf = 5
    assert rows_per_w % chunk == 0
    n_chunks = rows_per_w // chunk
    mesh = plsc.VectorSubcoreMesh(core_axis_name="c", subcore_axis_name="s")

    @functools.partial(
        pl.kernel,
        mesh=mesh,
        out_type=jax.ShapeDtypeStruct((n_rows, _DIM), jnp.float32),
        scratch_types=[
            pltpu.VMEM((rows_per_w,), jnp.int32),
            pltpu.VMEM((nbuf, chunk, _DIM), jnp.float32),
            [pltpu.SemaphoreType.DMA] * nbuf,
            [pltpu.SemaphoreType.DMA] * nbuf,
        ],
    )
    def gather(table_hbm, idx_hbm, out_hbm, idx_v, rows_v, gsems, ssems):
        wid = lax.axis_index("s") * info.num_cores + lax.axis_index("c")
        base = wid * rows_per_w
        pltpu.sync_copy(idx_hbm.at[pl.ds(base, rows_per_w)], idx_v)

        def g_copy(g, b):
            return pltpu.make_async_copy(
                table_hbm.at[idx_v.at[pl.ds(g * chunk, chunk)]],
                rows_v.at[b], gsems[b])

        def s_copy(g, b):
            return pltpu.make_async_copy(
                rows_v.at[b], out_hbm.at[pl.ds(base + g * chunk, chunk)],
                ssems[b])

        g_copy(0, 0).start()
        g_copy(0, 0).wait()
        for g in range(n_chunks):
            b = g % nbuf
            if g >= nbuf:
                s_copy(g - nbuf, b).wait()
            s_copy(g, b).start()
        for g in range(max(0, n_chunks - nbuf), n_chunks):
            s_copy(g, g % nbuf).wait()

    _cache["gather"] = gather
    return gather


def _make_tc_gather(kept):
    kept = [int(v) for v in kept]

    def body(in_ref, out_ref, sem):
        copies = [
            pltpu.make_async_copy(in_ref.at[:, src], out_ref.at[:, i], sem)
            for i, src in enumerate(kept)
        ]
        for c in copies:
            c.start()
        for c in copies:
            c.wait()

    return pl.pallas_call(
        body,
        in_specs=[pl.BlockSpec(memory_space=pl.ANY)],
        out_specs=pl.BlockSpec(memory_space=pl.ANY),
        out_shape=jax.ShapeDtypeStruct((_BATCH, _KEEP, _DIM), jnp.float32),
        scratch_shapes=[pltpu.SemaphoreType.DMA],
    )


def kernel(input):
    assert input.shape == (_BATCH, _SEQ, _DIM), input.shape
    fwd, bwd, flat_idx = _perm_consts()
    table = input.reshape(_BATCH * _SEQ, _DIM)
    out = _make_gather()(table, jnp.asarray(flat_idx))
    out = out.reshape(_BATCH, _KEEP, _DIM)
    return (out, jnp.asarray(fwd), jnp.asarray(bwd))
